# baseline (speedup 1.0000x reference)
import jax
import jax.numpy as jnp
from jax import lax
from jax.experimental import pallas as pl
from jax.experimental.pallas import tpu as pltpu
from jax.experimental.pallas import tpu_sc as plsc

NC = 2
NS = 16
L = 16
NW = NC * NS

BATCH = 16384
DIM = 32
HALF = DIM // 2
BPW = BATCH // NW
IPW = 2 * BPW
CHUNK = 128
NCHUNK = IPW // CHUNK


def _body(ju_hbm, ji_hbm, uw_hbm, iw_hbm, out_hbm,
          uidx_v, iidx_v, urows_v, irows_v, out_v, sem):
    wid = lax.axis_index("s") * NC + lax.axis_index("c")

    pltpu.sync_copy(ju_hbm.at[pl.ds(wid * NCHUNK, NCHUNK)], uidx_v)
    pltpu.sync_copy(ji_hbm.at[pl.ds(wid * NCHUNK, NCHUNK)], iidx_v)

    copies = []
    for j in range(NCHUNK):
        copies.append(pltpu.async_copy(
            uw_hbm.at[uidx_v.at[j]], urows_v.at[pl.ds(j * CHUNK, CHUNK)], sem))
        copies.append(pltpu.async_copy(
            iw_hbm.at[iidx_v.at[j]], irows_v.at[pl.ds(j * CHUNK, CHUNK)], sem))
    for c in copies:
        c.wait()

    lane = lax.iota(jnp.int32, L)

    def group(g, _):
        pe = (g * L + lane) * 2
        po = pe + 1
        acc = jnp.zeros((L,), jnp.float32)
        for d in range(HALF):
            col = jnp.full((L,), d, jnp.int32)
            acc = acc + (plsc.load_gather(urows_v, [pe, col])
                         * plsc.load_gather(irows_v, [pe, col]))
            acc = acc + (plsc.load_gather(urows_v, [po, col])
                         * plsc.load_gather(irows_v, [po, col]))
        out_v[pl.ds(pl.multiple_of(g * L, L), L)] = acc
        return _

    lax.fori_loop(0, BPW // L, group, 0)

    pltpu.sync_copy(out_v, out_hbm.at[pl.ds(wid * BPW, BPW)])


def kernel(u, i, user_weight, item_weight):
    u32 = u.astype(jnp.int32)
    i32 = i.astype(jnp.int32)
    ju = jnp.stack([u32 * 2, u32 * 2 + 1], axis=-1).reshape(-1)
    ji = jnp.stack([i32 * 2, i32 * 2 + 1], axis=-1).reshape(-1)
    ju2 = ju.reshape(2 * BATCH // CHUNK, CHUNK)
    ji2 = ji.reshape(2 * BATCH // CHUNK, CHUNK)
    uw2 = user_weight.reshape(-1, HALF)
    iw2 = item_weight.reshape(-1, HALF)
    mesh = plsc.VectorSubcoreMesh(core_axis_name="c", subcore_axis_name="s",
                                  num_cores=NC, num_subcores=NS)
    f = pl.kernel(
        _body,
        out_type=jax.ShapeDtypeStruct((BATCH,), jnp.float32),
        mesh=mesh,
        compiler_params=pltpu.CompilerParams(needs_layout_passes=False,
                                             use_tc_tiling_on_sc=False),
        scratch_types=[
            pltpu.VMEM((NCHUNK, CHUNK), jnp.int32),
            pltpu.VMEM((NCHUNK, CHUNK), jnp.int32),
            pltpu.VMEM((IPW, HALF), jnp.float32),
            pltpu.VMEM((IPW, HALF), jnp.float32),
            pltpu.VMEM((BPW,), jnp.float32),
            pltpu.SemaphoreType.DMA,
        ],
    )
    return f(ju2, ji2, uw2, iw2)


# speedup vs baseline: 3.8060x; 3.8060x over previous
import jax
import jax.numpy as jnp
from jax import lax
from jax.experimental import pallas as pl
from jax.experimental.pallas import tpu as pltpu
from jax.experimental.pallas import tpu_sc as plsc

NC = 2
NS = 16
L = 16
NW = NC * NS

BATCH = 16384
DIM = 32
G = 4
S = 8
TW = 128
RB = 8
BPW = BATCH // NW
NB = BPW // RB


def _body(u_hbm, i_hbm, wtu_hbm, wti_hbm, out_hbm,
          uidx_v, iidx_v, ublk, iblk, out_v, sem0):
    wid = lax.axis_index("s") * NC + lax.axis_index("c")
    base = wid * BPW

    pltpu.sync_copy(u_hbm.at[pl.ds(base, BPW)], uidx_v.at[pl.ds(0, BPW)])
    pltpu.sync_copy(i_hbm.at[pl.ds(base, BPW)], iidx_v.at[pl.ds(0, BPW)])

    lane = lax.iota(jnp.int32, L)
    lo_mask = lane < RB

    def batch(b, _):
        off = pl.multiple_of(b * RB, RB)
        ruv = uidx_v[pl.ds(off, L)]
        riv = iidx_v[pl.ds(off, L)]
        copies = []
        for j in range(RB):
            ru = pl.multiple_of(ruv[j] - (ruv[j] & (TW - 1)), TW)
            ri = pl.multiple_of(riv[j] - (riv[j] & (TW - 1)), TW)
            copies.append(pltpu.async_copy(
                wtu_hbm.at[:, :, pl.ds(ru, TW)], ublk.at[j], sem0))
            copies.append(pltpu.async_copy(
                wti_hbm.at[:, :, pl.ds(ri, TW)], iblk.at[j], sem0))
        for c in copies:
            c.wait()

        slotv = lane & (RB - 1)
        colu = ruv & (TW - 1)
        coli = riv & (TW - 1)
        acc = jnp.zeros((L,), jnp.float32)
        for gg in range(G):
            ggv = jnp.full((L,), gg, jnp.int32)
            for s in range(S):
                sv = jnp.full((L,), s, jnp.int32)
                acc = acc + (plsc.load_gather(ublk, [slotv, ggv, sv, colu])
                             * plsc.load_gather(iblk, [slotv, ggv, sv, coli]))
        plsc.store_compressed(out_v.at[pl.ds(off, L)], acc, mask=lo_mask)
        return _

    lax.fori_loop(0, NB, batch, 0)

    pltpu.sync_copy(out_v.at[pl.ds(0, BPW)], out_hbm.at[pl.ds(base, BPW)])


def kernel(u, i, user_weight, item_weight):
    u32 = u.astype(jnp.int32)
    i32 = i.astype(jnp.int32)
    wtu = user_weight.T.reshape(G, S, -1)
    wti = item_weight.T.reshape(G, S, -1)
    mesh = plsc.VectorSubcoreMesh(core_axis_name="c", subcore_axis_name="s",
                                  num_cores=NC, num_subcores=NS)
    f = pl.kernel(
        _body,
        out_type=jax.ShapeDtypeStruct((BATCH,), jnp.float32),
        mesh=mesh,
        compiler_params=pltpu.CompilerParams(needs_layout_passes=False,
                                             use_tc_tiling_on_sc=True),
        scratch_types=[
            pltpu.VMEM((BPW + L,), jnp.int32),
            pltpu.VMEM((BPW + L,), jnp.int32),
            pltpu.VMEM((RB, G, S, TW), jnp.float32),
            pltpu.VMEM((RB, G, S, TW), jnp.float32),
            pltpu.VMEM((BPW + RB,), jnp.float32),
            pltpu.SemaphoreType.DMA,
        ],
    )
    return f(u32, i32, wtu, wti)


# speedup vs baseline: 4.4770x; 1.1763x over previous
import jax
import jax.numpy as jnp
from jax import lax
from jax.experimental import pallas as pl
from jax.experimental.pallas import tpu as pltpu
from jax.experimental.pallas import tpu_sc as plsc

NC = 2
NS = 16
L = 16
NW = NC * NS

BATCH = 16384
DIM = 32
G = 4
S = 8
TW = 128
RS = 4
BPW = BATCH // NW
NSET = BPW // RS
PPW = 2 * BPW


def _body(u_hbm, i_hbm, wtu_hbm, wti_hbm, out_hbm,
          uidx_v, iidx_v, ublk, iblk, opad_v, out_v, sem0):
    wid = lax.axis_index("s") * NC + lax.axis_index("c")

    pltpu.sync_copy(u_hbm.at[pl.ds(wid * PPW, PPW)], uidx_v.at[pl.ds(0, PPW)])
    pltpu.sync_copy(i_hbm.at[pl.ds(wid * PPW, PPW)], iidx_v.at[pl.ds(0, PPW)])

    lane = lax.iota(jnp.int32, L)
    set_mask = lane < RS

    def fire(k, slot):
        off = pl.multiple_of(k * 2 * RS, 8)
        ruv = uidx_v[pl.ds(off, L)]
        riv = iidx_v[pl.ds(off, L)]
        for j in range(RS):
            ru = pl.multiple_of(ruv[j] - (ruv[j] & (TW - 1)), TW)
            ri = pl.multiple_of(riv[j] - (riv[j] & (TW - 1)), TW)
            pltpu.async_copy(wtu_hbm.at[:, :, pl.ds(ru, TW)],
                             ublk.at[slot, j], sem0)
            pltpu.async_copy(wti_hbm.at[:, :, pl.ds(ri, TW)],
                             iblk.at[slot, j], sem0)

    def step(k, _):
        @pl.when(k + 1 < NSET)
        def _fire_next():
            fire(k + 1, (k + 1) & 1)

        for _j in range(2 * RS):
            pltpu.make_async_copy(wtu_hbm.at[:, :, pl.ds(0, TW)],
                                  ublk.at[0, 0], sem0).wait()

        off = pl.multiple_of(k * 2 * RS, 8)
        ruv = uidx_v[pl.ds(off, L)]
        riv = iidx_v[pl.ds(off, L)]
        slot = k & 1
        slotv = jnp.zeros((L,), jnp.int32) + slot
        rowv = lane & (RS - 1)
        colu = ruv & (TW - 1)
        coli = riv & (TW - 1)
        acc = jnp.zeros((L,), jnp.float32)
        for gg in range(G):
            ggv = jnp.full((L,), gg, jnp.int32)
            for s in range(S):
                sv = jnp.full((L,), s, jnp.int32)
                acc = acc + (plsc.load_gather(ublk, [slotv, rowv, ggv, sv, colu])
                             * plsc.load_gather(iblk, [slotv, rowv, ggv, sv, coli]))
        plsc.store_compressed(opad_v.at[pl.ds(off, L)], acc, mask=set_mask)
        return _

    fire(0, 0)
    lax.fori_loop(0, NSET, step, 0)

    def compact(g, _):
        src = g * 2 * L + lax.shift_right_logical(lane, 2) * 2 * RS + (lane & (RS - 1))
        out_v[pl.ds(pl.multiple_of(g * L, L), L)] = plsc.load_gather(opad_v, [src])
        return _

    lax.fori_loop(0, BPW // L, compact, 0)

    pltpu.sync_copy(out_v, out_hbm.at[pl.ds(wid * BPW, BPW)])


def kernel(u, i, user_weight, item_weight):
    u32 = u.astype(jnp.int32)
    i32 = i.astype(jnp.int32)
    up = jnp.pad(u32.reshape(-1, RS), ((0, 0), (0, 8 - RS))).reshape(-1)
    ip = jnp.pad(i32.reshape(-1, RS), ((0, 0), (0, 8 - RS))).reshape(-1)
    wtu = user_weight.T.reshape(G, S, -1)
    wti = item_weight.T.reshape(G, S, -1)
    mesh = plsc.VectorSubcoreMesh(core_axis_name="c", subcore_axis_name="s",
                                  num_cores=NC, num_subcores=NS)
    f = pl.kernel(
        _body,
        out_type=jax.ShapeDtypeStruct((BATCH,), jnp.float32),
        mesh=mesh,
        compiler_params=pltpu.CompilerParams(needs_layout_passes=False,
                                             use_tc_tiling_on_sc=True),
        scratch_types=[
            pltpu.VMEM((PPW + 8,), jnp.int32),
            pltpu.VMEM((PPW + 8,), jnp.int32),
            pltpu.VMEM((2, RS, G, S, TW), jnp.float32),
            pltpu.VMEM((2, RS, G, S, TW), jnp.float32),
            pltpu.VMEM((PPW + 8,), jnp.float32),
            pltpu.VMEM((BPW,), jnp.float32),
            pltpu.SemaphoreType.DMA,
        ],
    )
    return f(up, ip, wtu, wti)


# speedup vs baseline: 4.4977x; 1.0046x over previous
import jax
import jax.numpy as jnp
from jax import lax
from jax.experimental import pallas as pl
from jax.experimental.pallas import tpu as pltpu
from jax.experimental.pallas import tpu_sc as plsc

NC = 2
NS = 16
L = 16
NW = NC * NS

BATCH = 16384
DIM = 32
G = 4
S = 8
TW = 128
RS = 4
NSLOT = 3
BPW = BATCH // NW
NSET = BPW // RS
PPW = 2 * BPW


def _body(u_hbm, i_hbm, wtu_hbm, wti_hbm, out_hbm,
          uidx_v, iidx_v, ublk, iblk, opad_v, out_v, sem0):
    wid = lax.axis_index("s") * NC + lax.axis_index("c")

    pltpu.sync_copy(u_hbm.at[pl.ds(wid * PPW, PPW)], uidx_v.at[pl.ds(0, PPW)])
    pltpu.sync_copy(i_hbm.at[pl.ds(wid * PPW, PPW)], iidx_v.at[pl.ds(0, PPW)])

    lane = lax.iota(jnp.int32, L)
    set_mask = lane < RS

    def fire(k, slot):
        off = pl.multiple_of(k * 2 * RS, 8)
        ruv = uidx_v[pl.ds(off, L)]
        riv = iidx_v[pl.ds(off, L)]
        for j in range(RS):
            ru = pl.multiple_of(ruv[j] - (ruv[j] & (TW - 1)), TW)
            ri = pl.multiple_of(riv[j] - (riv[j] & (TW - 1)), TW)
            pltpu.async_copy(wtu_hbm.at[:, :, pl.ds(ru, TW)],
                             ublk.at[slot, j], sem0)
            pltpu.async_copy(wti_hbm.at[:, :, pl.ds(ri, TW)],
                             iblk.at[slot, j], sem0)

    def step(k, _):
        @pl.when(k + 2 < NSET)
        def _fire_next():
            nxt = k + 2
            fire(nxt, nxt - (nxt // NSLOT) * NSLOT)

        for _j in range(2 * RS):
            pltpu.make_async_copy(wtu_hbm.at[:, :, pl.ds(0, TW)],
                                  ublk.at[0, 0], sem0).wait()

        off = pl.multiple_of(k * 2 * RS, 8)
        ruv = uidx_v[pl.ds(off, L)]
        riv = iidx_v[pl.ds(off, L)]
        slot = k - (k // NSLOT) * NSLOT
        slotv = jnp.zeros((L,), jnp.int32) + slot
        rowv = lane & (RS - 1)
        colu = ruv & (TW - 1)
        coli = riv & (TW - 1)
        acc = jnp.zeros((L,), jnp.float32)
        for gg in range(G):
            ggv = jnp.full((L,), gg, jnp.int32)
            for s in range(S):
                sv = jnp.full((L,), s, jnp.int32)
                acc = acc + (plsc.load_gather(ublk, [slotv, rowv, ggv, sv, colu])
                             * plsc.load_gather(iblk, [slotv, rowv, ggv, sv, coli]))
        plsc.store_compressed(opad_v.at[pl.ds(off, L)], acc, mask=set_mask)
        return _

    fire(0, 0)
    fire(1, 1)
    lax.fori_loop(0, NSET, step, 0)

    def compact(g, _):
        src = g * 2 * L + lax.shift_right_logical(lane, 2) * 2 * RS + (lane & (RS - 1))
        out_v[pl.ds(pl.multiple_of(g * L, L), L)] = plsc.load_gather(opad_v, [src])
        return _

    lax.fori_loop(0, BPW // L, compact, 0)

    pltpu.sync_copy(out_v, out_hbm.at[pl.ds(wid * BPW, BPW)])


def kernel(u, i, user_weight, item_weight):
    u32 = u.astype(jnp.int32)
    i32 = i.astype(jnp.int32)
    up = jnp.pad(u32.reshape(-1, RS), ((0, 0), (0, 8 - RS))).reshape(-1)
    ip = jnp.pad(i32.reshape(-1, RS), ((0, 0), (0, 8 - RS))).reshape(-1)
    wtu = user_weight.T.reshape(G, S, -1)
    wti = item_weight.T.reshape(G, S, -1)
    mesh = plsc.VectorSubcoreMesh(core_axis_name="c", subcore_axis_name="s",
                                  num_cores=NC, num_subcores=NS)
    f = pl.kernel(
        _body,
        out_type=jax.ShapeDtypeStruct((BATCH,), jnp.float32),
        mesh=mesh,
        compiler_params=pltpu.CompilerParams(needs_layout_passes=False,
                                             use_tc_tiling_on_sc=True),
        scratch_types=[
            pltpu.VMEM((PPW + 8,), jnp.int32),
            pltpu.VMEM((PPW + 8,), jnp.int32),
            pltpu.VMEM((NSLOT, RS, G, S, TW), jnp.float32),
            pltpu.VMEM((NSLOT, RS, G, S, TW), jnp.float32),
            pltpu.VMEM((PPW + 8,), jnp.float32),
            pltpu.VMEM((BPW,), jnp.float32),
            pltpu.SemaphoreType.DMA,
        ],
    )
    return f(up, ip, wtu, wti)


# speedup vs baseline: 5.5243x; 1.2282x over previous
import jax
import jax.numpy as jnp
from jax import lax
from jax.experimental import pallas as pl
from jax.experimental.pallas import tpu as pltpu
from jax.experimental.pallas import tpu_sc as plsc

NC = 2
NS = 16
L = 16
NW = NC * NS

BATCH = 16384
DIM = 32
G = 4
S = 8
TW = 128
RS = 4
NSLOT = 3
BPW = BATCH // NW
NSET = BPW // RS
PPW = 2 * BPW


def _body(u_hbm, i_hbm, wtu_hbm, wti_hbm, out_hbm,
          uidx_v, iidx_v, ublk, iblk, opad_v, out_v, sem0):
    wid = lax.axis_index("s") * NC + lax.axis_index("c")

    pltpu.sync_copy(u_hbm.at[pl.ds(wid * PPW, PPW)], uidx_v.at[pl.ds(0, PPW)])
    pltpu.sync_copy(i_hbm.at[pl.ds(wid * PPW, PPW)], iidx_v.at[pl.ds(0, PPW)])

    lane = lax.iota(jnp.int32, L)
    set_mask = lane < RS

    def fire(k, slot):
        off = pl.multiple_of(k * 2 * RS, 8)
        ruv = uidx_v[pl.ds(off, L)]
        riv = iidx_v[pl.ds(off, L)]
        for j in range(RS):
            for src, idx, blk in ((wtu_hbm, ruv, ublk), (wti_hbm, riv, iblk)):
                r = idx[j]
                rt = pl.multiple_of(r - (r & (TW - 1)), TW)
                hi = (r & (TW // 2)) != 0

                @pl.when(hi)
                def _full(src=src, blk=blk, rt=rt, j=j):
                    pltpu.async_copy(src.at[:, :, pl.ds(rt, TW)],
                                     blk.at[slot, j], sem0)

                @pl.when(jnp.logical_not(hi))
                def _half(src=src, blk=blk, rt=rt, j=j):
                    pltpu.async_copy(
                        src.at[:, :, pl.ds(rt, TW // 2)],
                        blk.at[slot, j, :, :, pl.ds(0, TW // 2)], sem0)

    def step(k, _):
        @pl.when(k + 2 < NSET)
        def _fire_next():
            nxt = k + 2
            fire(nxt, nxt - (nxt // NSLOT) * NSLOT)

        doff = pl.multiple_of(k * 2 * RS, 8)
        druv = uidx_v[pl.ds(doff, L)]
        driv = iidx_v[pl.ds(doff, L)]
        for j in range(RS):
            for idx in (druv, driv):
                hi = (idx[j] & (TW // 2)) != 0

                @pl.when(hi)
                def _wfull():
                    pltpu.make_async_copy(wtu_hbm.at[:, :, pl.ds(0, TW)],
                                          ublk.at[0, 0], sem0).wait()

                @pl.when(jnp.logical_not(hi))
                def _whalf():
                    pltpu.make_async_copy(
                        wtu_hbm.at[:, :, pl.ds(0, TW // 2)],
                        ublk.at[0, 0, :, :, pl.ds(0, TW // 2)], sem0).wait()

        off = pl.multiple_of(k * 2 * RS, 8)
        ruv = uidx_v[pl.ds(off, L)]
        riv = iidx_v[pl.ds(off, L)]
        slot = k - (k // NSLOT) * NSLOT
        slotv = jnp.zeros((L,), jnp.int32) + slot
        rowv = lane & (RS - 1)
        colu = ruv & (TW - 1)
        coli = riv & (TW - 1)
        acc = jnp.zeros((L,), jnp.float32)
        for gg in range(G):
            ggv = jnp.full((L,), gg, jnp.int32)
            for s in range(S):
                sv = jnp.full((L,), s, jnp.int32)
                acc = acc + (plsc.load_gather(ublk, [slotv, rowv, ggv, sv, colu])
                             * plsc.load_gather(iblk, [slotv, rowv, ggv, sv, coli]))
        plsc.store_compressed(opad_v.at[pl.ds(off, L)], acc, mask=set_mask)
        return _

    fire(0, 0)
    fire(1, 1)
    lax.fori_loop(0, NSET, step, 0)

    def compact(g, _):
        src = g * 2 * L + lax.shift_right_logical(lane, 2) * 2 * RS + (lane & (RS - 1))
        out_v[pl.ds(pl.multiple_of(g * L, L), L)] = plsc.load_gather(opad_v, [src])
        return _

    lax.fori_loop(0, BPW // L, compact, 0)

    pltpu.sync_copy(out_v, out_hbm.at[pl.ds(wid * BPW, BPW)])


def kernel(u, i, user_weight, item_weight):
    u32 = u.astype(jnp.int32)
    i32 = i.astype(jnp.int32)
    up = jnp.pad(u32.reshape(-1, RS), ((0, 0), (0, 8 - RS))).reshape(-1)
    ip = jnp.pad(i32.reshape(-1, RS), ((0, 0), (0, 8 - RS))).reshape(-1)
    wtu = user_weight.T.reshape(G, S, -1)
    wti = item_weight.T.reshape(G, S, -1)
    mesh = plsc.VectorSubcoreMesh(core_axis_name="c", subcore_axis_name="s",
                                  num_cores=NC, num_subcores=NS)
    f = pl.kernel(
        _body,
        out_type=jax.ShapeDtypeStruct((BATCH,), jnp.float32),
        mesh=mesh,
        compiler_params=pltpu.CompilerParams(needs_layout_passes=False,
                                             use_tc_tiling_on_sc=True),
        scratch_types=[
            pltpu.VMEM((PPW + 8,), jnp.int32),
            pltpu.VMEM((PPW + 8,), jnp.int32),
            pltpu.VMEM((NSLOT, RS, G, S, TW), jnp.float32),
            pltpu.VMEM((NSLOT, RS, G, S, TW), jnp.float32),
            pltpu.VMEM((PPW + 8,), jnp.float32),
            pltpu.VMEM((BPW,), jnp.float32),
            pltpu.SemaphoreType.DMA,
        ],
    )
    return f(up, ip, wtu, wti)
